# baseline (speedup 1.0000x reference)
import functools

import jax
import jax.numpy as jnp
from jax import lax
from jax.experimental import pallas as pl
from jax.experimental.pallas import tpu as pltpu
from jax.experimental.pallas import tpu_sc as plsc


def _largest_divisor_leq(n, cap, multiple_of=1):
    for d in range(cap, 0, -1):
        if n % d == 0 and d % multiple_of == 0:
            return d
    return 1


@functools.lru_cache(maxsize=None)
def _make_seg_sum(N, E, D, fused):
    info = plsc.get_sparse_core_info()
    NC, NS = info.num_cores, info.num_subcores
    NW = NC * NS
    CH = 128
    assert E % (NW * CH) == 0, (E, NW, CH)
    EPW = E // NW
    NST = EPW // CH
    RPS = N // NS
    assert N % NS == 0, (N, NS)
    ZCH = _largest_divisor_leq(RPS, CH)
    K = 8
    assert NST % K == 0 and NST > K, (NST, K)
    NA = N + 16

    mesh = plsc.VectorSubcoreMesh(core_axis_name="c", subcore_axis_name="s")

    if fused:
        out_type = jax.ShapeDtypeStruct((N, 128), jnp.float32)
    else:
        out_type = jax.ShapeDtypeStruct((NC, N, D), jnp.float32)

    scratch = [
        pltpu.VMEM((NST, CH), jnp.int32),
        pltpu.VMEM((NST, CH), jnp.int32),
        pltpu.VMEM((K, CH, D), jnp.float32),
        pltpu.VMEM_SHARED((N, D), jnp.float32),
        pltpu.VMEM_SHARED((NA, D), jnp.float32),
    ]
    if fused:
        scratch = scratch + [
            pltpu.VMEM((2, ZCH, D), jnp.float32),
            pltpu.VMEM((2, ZCH, D), jnp.float32),
            pltpu.VMEM((2, ZCH, D), jnp.float32),
        ]
    scratch = scratch + [pltpu.SemaphoreType.DMA] * (2 * K)

    def body(*refs):
        if fused:
            (p_in, xw_in, edge3d, out, src_v, dst_v, bufs, tab_sh,
             acc, pb0, pb1, xb) = refs[:12]
            sems = refs[12:]
        else:
            (xw_in, edge3d, out, src_v, dst_v, bufs, tab_sh, acc) = refs[:8]
            sems = refs[8:]
        sem_g = sems[:K]
        sem_s = sems[K:]
        cid = lax.axis_index("c")
        sid = lax.axis_index("s")
        wid = sid * NC + cid
        r0 = sid * RPS

        pltpu.async_copy(edge3d.at[0, pl.ds(wid * NST, NST)], src_v, sem_s[0])
        pltpu.async_copy(edge3d.at[1, pl.ds(wid * NST, NST)], dst_v, sem_s[1])
        if not fused:
            pltpu.async_copy(xw_in.at[pl.ds(r0, RPS), pl.ds(0, D)],
                             tab_sh.at[pl.ds(r0, RPS)], sem_s[2])

        zero = jnp.zeros((16,), jnp.float32)

        def zfill(i, carry):
            for g in range(D // 16):
                bufs[0, i, pl.ds(g * 16, 16)] = zero
            return carry

        lax.fori_loop(0, ZCH, zfill, 0)
        for k in range(RPS // ZCH):
            pltpu.sync_copy(bufs.at[0, pl.ds(0, ZCH)],
                            acc.at[pl.ds(r0 + k * ZCH, ZCH)])

        if fused:
            U = 5
            assert ZCH % U == 0
            NK = RPS // ZCH

            def stage_load(k, b):
                rb = r0 + k * ZCH
                pltpu.async_copy(p_in.at[0, pl.ds(rb, ZCH)], pb0.at[b],
                                 sem_g[0])
                pltpu.async_copy(p_in.at[1, pl.ds(rb, ZCH)], pb1.at[b],
                                 sem_g[1])
                pltpu.async_copy(xw_in.at[pl.ds(rb, ZCH), pl.ds(D, D)],
                                 xb.at[b], sem_g[2])

            def stage_wait(b):
                pltpu.make_async_copy(p_in.at[0, pl.ds(r0, ZCH)], pb0.at[b],
                                      sem_g[0]).wait()
                pltpu.make_async_copy(p_in.at[1, pl.ds(r0, ZCH)], pb1.at[b],
                                      sem_g[1]).wait()
                pltpu.make_async_copy(p_in.at[0, pl.ds(r0, ZCH)], xb.at[b],
                                      sem_g[2]).wait()

            stage_load(0, 0)
            for k in range(NK):
                b = k % 2
                rb = r0 + k * ZCH
                stage_wait(b)
                if k + 1 < NK:
                    stage_load(k + 1, 1 - b)

                def ebody(i, carry, b=b):
                    for u in range(U):
                        r = i * U + u
                        for g in range(D // 16):
                            sl = pl.ds(g * 16, 16)
                            v = jnp.maximum(pb0[b, r, sl] + pb1[b, r, sl]
                                            + xb[b, r, sl], 0.0)
                            pb0[b, r, sl] = v
                    return carry

                lax.fori_loop(0, ZCH // U, ebody, 0)
                pltpu.sync_copy(pb0.at[b], tab_sh.at[pl.ds(rb, ZCH)])

                @pl.when(cid == 0)
                def _write_h():
                    pltpu.sync_copy(pb0.at[b],
                                    out.at[pl.ds(rb, ZCH), pl.ds(2 * D, D)])
        else:
            pltpu.make_async_copy(xw_in.at[pl.ds(r0, RPS), pl.ds(0, D)],
                                  tab_sh.at[pl.ds(r0, RPS)], sem_s[2]).wait()

        pltpu.make_async_copy(edge3d.at[0, pl.ds(wid * NST, NST)], src_v,
                              sem_s[0]).wait()
        pltpu.make_async_copy(edge3d.at[1, pl.ds(wid * NST, NST)], dst_v,
                              sem_s[1]).wait()
        plsc.subcore_barrier()

        def gsrc(i):
            return tab_sh.at[i]

        PF = K // 2
        for j in range(PF):
            pltpu.async_copy(gsrc(src_v.at[j]), bufs.at[j], sem_g[j])

        def chunk_step(c, j):
            pltpu.make_async_copy(gsrc(src_v.at[c]), bufs.at[j],
                                  sem_g[j]).wait()
            pltpu.async_copy(bufs.at[j], acc.at[dst_v.at[c]],
                             sem_s[j], add=True)
            jp = (j + PF) % K
            cp = c + PF

            @pl.when(cp < NST)
            def _prefetch():
                @pl.when(c >= K - PF)
                def _wait_scatter():
                    pltpu.make_async_copy(bufs.at[jp], acc.at[dst_v.at[c]],
                                          sem_s[jp]).wait()
                pltpu.async_copy(gsrc(src_v.at[cp]), bufs.at[jp],
                                 sem_g[jp])

        def loop_body(i, carry):
            for j in range(K):
                chunk_step(i * K + j, j)
            return carry

        lax.fori_loop(0, NST // K, loop_body, 0)
        for j in range(K):
            pltpu.make_async_copy(bufs.at[j], acc.at[dst_v.at[NST - K + j]],
                                  sem_s[j]).wait()

        plsc.subcore_barrier()
        if fused:
            pltpu.sync_copy(acc.at[pl.ds(r0, RPS)],
                            out.at[pl.ds(r0, RPS), pl.ds(cid * D, D)])
        else:
            pltpu.sync_copy(acc.at[pl.ds(r0, RPS)],
                            out.at[cid, pl.ds(r0, RPS)])

    ealign = NW * CH * K
    return functools.partial(
        pl.kernel, body,
        out_type=out_type,
        mesh=mesh,
        compiler_params=pltpu.CompilerParams(use_tc_tiling_on_sc=False),
        scratch_types=scratch,
    )(), ealign


def _tc1_body(x_ref, w_ref, b_ref, xw_ref):
    z = jnp.dot(x_ref[...], w_ref[...], preferred_element_type=jnp.float32)
    zb = z + b_ref[...]
    D2 = zb.shape[1]
    xw_ref[:, 0:D2] = zb


def _tc3_body(combo_ref, wl_ref, wr_ref, b_ref, o_ref, *, D):
    blk = combo_ref[...]
    a = blk[:, 0:D] + blk[:, D:2 * D]
    h = blk[:, 2 * D:3 * D]
    z = jnp.dot(a, wl_ref[...], preferred_element_type=jnp.float32)
    z = z + jnp.dot(h, wr_ref[...], preferred_element_type=jnp.float32)
    z = z + b_ref[...]
    m = jnp.max(z, axis=1, keepdims=True)
    s = jnp.sum(jnp.exp(z - m), axis=1, keepdims=True)
    o_ref[...] = z - m - jnp.log(s)


def kernel(x, edge_index, W1l, b1, W1r, W2l, b2, W2r):
    f32 = jnp.float32
    N, D_IN = x.shape
    E = edge_index.shape[1]
    D = W1l.shape[1]
    C = W2l.shape[1]

    info = plsc.get_sparse_core_info()
    ealign = info.num_cores * info.num_subcores * 128 * 4
    E_pad = -(-E // ealign) * ealign
    seg_plain, _ = _make_seg_sum(N, E_pad, D, False)
    seg_fused, _ = _make_seg_sum(N, E_pad, D, True)

    pad = E_pad - E
    if pad:
        filler = jnp.broadcast_to(
            jnp.array([[0], [N]], dtype=edge_index.dtype), (2, pad))
        edge_full = jnp.concatenate([edge_index, filler], axis=1)
    else:
        edge_full = edge_index
    edge3d = edge_full.reshape(2, E_pad // 128, 128)

    R = _largest_divisor_leq(N, 2048, multiple_of=8)
    nblk = N // R

    wcat = jnp.concatenate([W1l, W1r], axis=1)
    bcat = jnp.concatenate([jnp.zeros((D,), f32), b1]).reshape(1, 2 * D)
    xw = pl.pallas_call(
        _tc1_body,
        grid=(nblk,),
        in_specs=[
            pl.BlockSpec((R, D_IN), lambda i: (i, 0)),
            pl.BlockSpec((D_IN, 2 * D), lambda i: (0, 0)),
            pl.BlockSpec((1, 2 * D), lambda i: (0, 0)),
        ],
        out_specs=pl.BlockSpec((R, 128), lambda i: (i, 0)),
        out_shape=jax.ShapeDtypeStruct((N, 128), f32),
    )(x, wcat, bcat)

    p = seg_plain(xw, edge3d)
    combo = seg_fused(p, xw, edge3d)

    out = pl.pallas_call(
        functools.partial(_tc3_body, D=D),
        grid=(nblk,),
        in_specs=[
            pl.BlockSpec((R, 128), lambda i: (i, 0)),
            pl.BlockSpec((D, C), lambda i: (0, 0)),
            pl.BlockSpec((D, C), lambda i: (0, 0)),
            pl.BlockSpec((1, C), lambda i: (0, 0)),
        ],
        out_specs=pl.BlockSpec((R, C), lambda i: (i, 0)),
        out_shape=jax.ShapeDtypeStruct((N, C), f32),
    )(combo, W2l, W2r, b2.reshape(1, C))

    return out


# speedup vs baseline: 1.0132x; 1.0132x over previous
import functools

import jax
import jax.numpy as jnp
from jax import lax
from jax.experimental import pallas as pl
from jax.experimental.pallas import tpu as pltpu
from jax.experimental.pallas import tpu_sc as plsc


def _largest_divisor_leq(n, cap, multiple_of=1):
    for d in range(cap, 0, -1):
        if n % d == 0 and d % multiple_of == 0:
            return d
    return 1


@functools.lru_cache(maxsize=None)
def _make_seg_sum(N, E, D, fused):
    info = plsc.get_sparse_core_info()
    NC, NS = info.num_cores, info.num_subcores
    NW = NC * NS
    CH = 128
    assert E % (NW * CH) == 0, (E, NW, CH)
    EPW = E // NW
    NST = EPW // CH
    RPS = N // NS
    assert N % NS == 0, (N, NS)
    ZCH = _largest_divisor_leq(RPS, CH)
    K = 8
    assert NST % K == 0 and NST > K, (NST, K)
    NA = N + 16

    mesh = plsc.VectorSubcoreMesh(core_axis_name="c", subcore_axis_name="s")

    if fused:
        out_type = jax.ShapeDtypeStruct((N, 128), jnp.float32)
    else:
        out_type = jax.ShapeDtypeStruct((NC, N, D), jnp.float32)

    scratch = [
        pltpu.VMEM((NST, CH), jnp.int32),
        pltpu.VMEM((NST, CH), jnp.int32),
        pltpu.VMEM((K, CH, D), jnp.float32),
        pltpu.VMEM_SHARED((N, D), jnp.float32),
        pltpu.VMEM_SHARED((NA, D), jnp.float32),
    ]
    if fused:
        scratch = scratch + [
            pltpu.VMEM((2, ZCH, D), jnp.float32),
            pltpu.VMEM((2, ZCH, D), jnp.float32),
            pltpu.VMEM((2, ZCH, D), jnp.float32),
        ]
    scratch = scratch + [pltpu.SemaphoreType.DMA] * (2 * K)

    def body(*refs):
        if fused:
            (p_in, xw_in, edge3d, out, src_v, dst_v, bufs, tab_sh,
             acc, pb0, pb1, xb) = refs[:12]
            sems = refs[12:]
        else:
            (xw_in, edge3d, out, src_v, dst_v, bufs, tab_sh, acc) = refs[:8]
            sems = refs[8:]
        sem_g = sems[:K]
        sem_s = sems[K:]
        cid = lax.axis_index("c")
        sid = lax.axis_index("s")
        wid = sid * NC + cid
        r0 = sid * RPS

        pltpu.async_copy(edge3d.at[0, pl.ds(wid * NST, NST)], src_v, sem_s[0])
        pltpu.async_copy(edge3d.at[1, pl.ds(wid * NST, NST)], dst_v, sem_s[1])
        if not fused:
            pltpu.async_copy(xw_in.at[pl.ds(r0, RPS), pl.ds(0, D)],
                             tab_sh.at[pl.ds(r0, RPS)], sem_s[2])

        zero = jnp.zeros((16,), jnp.float32)

        def zfill(i, carry):
            for g in range(D // 16):
                bufs[0, i, pl.ds(g * 16, 16)] = zero
            return carry

        lax.fori_loop(0, ZCH, zfill, 0)
        for k in range(RPS // ZCH):
            pltpu.sync_copy(bufs.at[0, pl.ds(0, ZCH)],
                            acc.at[pl.ds(r0 + k * ZCH, ZCH)])

        if fused:
            U = 5
            assert ZCH % U == 0
            NK = RPS // ZCH

            def stage_load(k, b):
                rb = r0 + k * ZCH
                pltpu.async_copy(p_in.at[0, pl.ds(rb, ZCH)], pb0.at[b],
                                 sem_g[0])
                pltpu.async_copy(p_in.at[1, pl.ds(rb, ZCH)], pb1.at[b],
                                 sem_g[1])
                pltpu.async_copy(xw_in.at[pl.ds(rb, ZCH), pl.ds(D, D)],
                                 xb.at[b], sem_g[2])

            def stage_wait(b):
                pltpu.make_async_copy(p_in.at[0, pl.ds(r0, ZCH)], pb0.at[b],
                                      sem_g[0]).wait()
                pltpu.make_async_copy(p_in.at[1, pl.ds(r0, ZCH)], pb1.at[b],
                                      sem_g[1]).wait()
                pltpu.make_async_copy(p_in.at[0, pl.ds(r0, ZCH)], xb.at[b],
                                      sem_g[2]).wait()

            stage_load(0, 0)
            for k in range(NK):
                b = k % 2
                rb = r0 + k * ZCH
                stage_wait(b)
                if k + 1 < NK:
                    stage_load(k + 1, 1 - b)

                def ebody(i, carry, b=b):
                    for u in range(U):
                        r = i * U + u
                        for g in range(D // 16):
                            sl = pl.ds(g * 16, 16)
                            v = jnp.maximum(pb0[b, r, sl] + pb1[b, r, sl]
                                            + xb[b, r, sl], 0.0)
                            pb0[b, r, sl] = v
                    return carry

                lax.fori_loop(0, ZCH // U, ebody, 0)
                pltpu.sync_copy(pb0.at[b], tab_sh.at[pl.ds(rb, ZCH)])

                @pl.when(cid == 0)
                def _write_h():
                    pltpu.sync_copy(pb0.at[b],
                                    out.at[pl.ds(rb, ZCH), pl.ds(2 * D, D)])
        else:
            pltpu.make_async_copy(xw_in.at[pl.ds(r0, RPS), pl.ds(0, D)],
                                  tab_sh.at[pl.ds(r0, RPS)], sem_s[2]).wait()

        pltpu.make_async_copy(edge3d.at[0, pl.ds(wid * NST, NST)], src_v,
                              sem_s[0]).wait()
        pltpu.make_async_copy(edge3d.at[1, pl.ds(wid * NST, NST)], dst_v,
                              sem_s[1]).wait()
        plsc.subcore_barrier()

        def gsrc(i):
            return tab_sh.at[i]

        PF = K - 1
        for j in range(PF):
            pltpu.async_copy(gsrc(src_v.at[j]), bufs.at[j], sem_g[j])

        def chunk_step(c, j):
            pltpu.make_async_copy(gsrc(src_v.at[c]), bufs.at[j],
                                  sem_g[j]).wait()
            pltpu.async_copy(bufs.at[j], acc.at[dst_v.at[c]],
                             sem_s[j], add=True)
            jp = (j + PF) % K
            cp = c + PF

            @pl.when(cp < NST)
            def _prefetch():
                @pl.when(c > 0)
                def _wait_scatter():
                    pltpu.make_async_copy(bufs.at[jp], acc.at[dst_v.at[c]],
                                          sem_s[jp]).wait()
                pltpu.async_copy(gsrc(src_v.at[cp]), bufs.at[jp],
                                 sem_g[jp])

        def loop_body(i, carry):
            for j in range(K):
                chunk_step(i * K + j, j)
            return carry

        lax.fori_loop(0, NST // K, loop_body, 0)
        for j in range(K):
            pltpu.make_async_copy(bufs.at[j], acc.at[dst_v.at[NST - K + j]],
                                  sem_s[j]).wait()

        plsc.subcore_barrier()
        if fused:
            pltpu.sync_copy(acc.at[pl.ds(r0, RPS)],
                            out.at[pl.ds(r0, RPS), pl.ds(cid * D, D)])
        else:
            pltpu.sync_copy(acc.at[pl.ds(r0, RPS)],
                            out.at[cid, pl.ds(r0, RPS)])

    ealign = NW * CH * K
    return functools.partial(
        pl.kernel, body,
        out_type=out_type,
        mesh=mesh,
        compiler_params=pltpu.CompilerParams(use_tc_tiling_on_sc=False),
        scratch_types=scratch,
    )(), ealign


def _tc1_body(x_ref, w_ref, b_ref, xw_ref):
    z = jnp.dot(x_ref[...], w_ref[...], preferred_element_type=jnp.float32)
    zb = z + b_ref[...]
    D2 = zb.shape[1]
    xw_ref[:, 0:D2] = zb


def _tc3_body(combo_ref, wl_ref, wr_ref, b_ref, o_ref, *, D):
    blk = combo_ref[...]
    a = blk[:, 0:D] + blk[:, D:2 * D]
    h = blk[:, 2 * D:3 * D]
    z = jnp.dot(a, wl_ref[...], preferred_element_type=jnp.float32)
    z = z + jnp.dot(h, wr_ref[...], preferred_element_type=jnp.float32)
    z = z + b_ref[...]
    m = jnp.max(z, axis=1, keepdims=True)
    s = jnp.sum(jnp.exp(z - m), axis=1, keepdims=True)
    o_ref[...] = z - m - jnp.log(s)


def kernel(x, edge_index, W1l, b1, W1r, W2l, b2, W2r):
    f32 = jnp.float32
    N, D_IN = x.shape
    E = edge_index.shape[1]
    D = W1l.shape[1]
    C = W2l.shape[1]

    info = plsc.get_sparse_core_info()
    ealign = info.num_cores * info.num_subcores * 128 * 4
    E_pad = -(-E // ealign) * ealign
    seg_plain, _ = _make_seg_sum(N, E_pad, D, False)
    seg_fused, _ = _make_seg_sum(N, E_pad, D, True)

    pad = E_pad - E
    if pad:
        filler = jnp.broadcast_to(
            jnp.array([[0], [N]], dtype=edge_index.dtype), (2, pad))
        edge_full = jnp.concatenate([edge_index, filler], axis=1)
    else:
        edge_full = edge_index
    edge3d = edge_full.reshape(2, E_pad // 128, 128)

    R = _largest_divisor_leq(N, 2048, multiple_of=8)
    nblk = N // R

    wcat = jnp.concatenate([W1l, W1r], axis=1)
    bcat = jnp.concatenate([jnp.zeros((D,), f32), b1]).reshape(1, 2 * D)
    xw = pl.pallas_call(
        _tc1_body,
        grid=(nblk,),
        in_specs=[
            pl.BlockSpec((R, D_IN), lambda i: (i, 0)),
            pl.BlockSpec((D_IN, 2 * D), lambda i: (0, 0)),
            pl.BlockSpec((1, 2 * D), lambda i: (0, 0)),
        ],
        out_specs=pl.BlockSpec((R, 128), lambda i: (i, 0)),
        out_shape=jax.ShapeDtypeStruct((N, 128), f32),
    )(x, wcat, bcat)

    p = seg_plain(xw, edge3d)
    combo = seg_fused(p, xw, edge3d)

    out = pl.pallas_call(
        functools.partial(_tc3_body, D=D),
        grid=(nblk,),
        in_specs=[
            pl.BlockSpec((R, 128), lambda i: (i, 0)),
            pl.BlockSpec((D, C), lambda i: (0, 0)),
            pl.BlockSpec((D, C), lambda i: (0, 0)),
            pl.BlockSpec((1, C), lambda i: (0, 0)),
        ],
        out_specs=pl.BlockSpec((R, C), lambda i: (i, 0)),
        out_shape=jax.ShapeDtypeStruct((N, C), f32),
    )(combo, W2l, W2r, b2.reshape(1, C))

    return out
